# Initial kernel scaffold; baseline (speedup 1.0000x reference)
import jax
import jax.numpy as jnp
from jax.experimental import pallas as pl


def kernel(x, adj, src, tgt, Msrc, Mtgt, Mgraph, params):
    raise NotImplementedError("write your pallas kernel here")


# speedup vs baseline: 3.6830x; 3.6830x over previous
import functools

import jax
import jax.numpy as jnp
from jax import lax
from jax.experimental import pallas as pl
from jax.experimental.pallas import tpu as pltpu
from jax.experimental.pallas import tpu_sc as plsc

N = 2048
E = 16384
G = 16
EPS = 1e-6
F32 = jnp.float32


def _dot(a, b, dims):
    return lax.dot_general(a, b, (dims, ((), ())), preferred_element_type=F32)


def _fold_s(S):
    m1 = jnp.max(S[0:1, :]) + jnp.max(S[2:3, :])
    m2 = jnp.max(S[1:2, :]) + jnp.max(S[3:4, :])
    return jnp.concatenate(
        [S[0:1] - m1, S[1:2] - m2, S[2:4], jnp.zeros((4, N), F32)], axis=0)


def _proj0_body(x_ref, wu_ref, bu_ref, wv_ref, ws_ref, u_out, v_out, s_out):
    x = x_ref[...]
    u_out[...] = _dot(wu_ref[...], x, ((1,), (1,))) + bu_ref[...][:, None]
    v_out[...] = _dot(wv_ref[...], x, ((1,), (1,)))
    s_out[...] = _fold_s(_dot(ws_ref[...], x, ((1,), (1,))))


def _proj_mid_body(num_ref, den_ref, wu_ref, bu_ref, wv_ref, ws_ref,
                   u_out, v_out, s_out):
    num = num_ref[...]
    den = den_ref[...]
    half = num.shape[0] // 2
    hc = jnp.concatenate([num[:half] / (den[0:1] + EPS),
                          num[half:] / (den[1:2] + EPS)], axis=0)
    u_out[...] = _dot(wu_ref[...], hc, ((1,), (0,))) + bu_ref[...][:, None]
    v_out[...] = _dot(wv_ref[...], hc, ((1,), (0,)))
    s_out[...] = _fold_s(_dot(ws_ref[...], hc, ((1,), (0,))))


def _final_body(num_ref, den_ref, mg_ref, w1_ref, b1_ref, w2_ref, b2_ref,
                out_ref):
    num = num_ref[...]
    den = den_ref[...]
    half = num.shape[0] // 2
    hc = jnp.concatenate([num[:half] / (den[0:1] + EPS),
                          num[half:] / (den[1:2] + EPS)], axis=0)
    pooled = _dot(mg_ref[...], hc, ((0,), (1,)))
    z1 = jax.nn.relu(_dot(pooled, w1_ref[...], ((1,), (1,)))
                     + b1_ref[...][None, :])
    out_ref[...] = _dot(z1, w2_ref[...], ((1,), (1,))) + b2_ref[...][None, :]


def _tc_call(body, out_shapes, args):
    return pl.pallas_call(
        body,
        out_shape=[jax.ShapeDtypeStruct(s, F32) for s in out_shapes],
    )(*args)


@functools.cache
def _make_sc_edge(C, CPW):
    info = plsc.get_sparse_core_info()
    NC, NS = info.num_cores, info.num_subcores
    NW = NC * NS
    assert C == CPW * NW
    mesh = plsc.VectorSubcoreMesh(core_axis_name="c", subcore_axis_name="s")

    @functools.partial(
        pl.kernel, mesh=mesh,
        compiler_params=pltpu.CompilerParams(needs_layout_passes=False),
        out_type=[jax.ShapeDtypeStruct((C * N,), F32),
                  jax.ShapeDtypeStruct((2 * N,), F32)],
        scratch_types=[
            pltpu.VMEM((CPW * N,), F32),
            pltpu.VMEM((CPW * N,), F32),
            pltpu.VMEM((N,), F32),
            pltpu.VMEM((N,), F32),
            pltpu.VMEM((CPW * N,), F32),
            pltpu.VMEM((N,), F32),
            pltpu.VMEM((E,), jnp.int32),
            pltpu.VMEM((E,), jnp.int32),
        ],
    )
    def sc_edge(u_hbm, v_hbm, s_hbm, src_hbm, tgt_hbm, num_out, den_out,
                u_v, v_v, p_v, q_v, num_v, den_v, src_v, tgt_v):
        wid = lax.axis_index("s") * NC + lax.axis_index("c")
        head = wid // (NW // 2)
        r0 = pl.multiple_of(wid * (CPW * N), CPW * N)

        pltpu.sync_copy(u_hbm.at[pl.ds(r0, CPW * N)], u_v)
        pltpu.sync_copy(v_hbm.at[pl.ds(r0, CPW * N)], v_v)
        pltpu.sync_copy(s_hbm.at[pl.ds(pl.multiple_of(head * N, N), N)], p_v)
        pltpu.sync_copy(
            s_hbm.at[pl.ds(pl.multiple_of((2 + head) * N, N), N)], q_v)
        pltpu.sync_copy(src_hbm, src_v)
        pltpu.sync_copy(tgt_hbm, tgt_v)

        zf = jnp.zeros((16,), F32)

        def zero_num(j, carry):
            num_v[pl.ds(pl.multiple_of(j * 16, 16), 16)] = zf
            return carry

        def zero_den(j, carry):
            den_v[pl.ds(pl.multiple_of(j * 16, 16), 16)] = zf
            return carry

        lax.fori_loop(0, CPW * N // 16, zero_num, 0)
        lax.fori_loop(0, N // 16, zero_den, 0)

        def edge_body(g, carry):
            base = pl.multiple_of(g * 16, 16)
            s16 = src_v[pl.ds(base, 16)]
            t16 = tgt_v[pl.ds(base, 16)]
            ps = plsc.load_gather(p_v, [s16])
            qt = plsc.load_gather(q_v, [t16])
            w = jnp.exp(ps + qt)
            plsc.addupdate_scatter(den_v, [t16], w)
            for c in range(CPW):
                us = plsc.load_gather(u_v, [s16 + (c * N)])
                vt = plsc.load_gather(v_v, [t16 + (c * N)])
                y = jnp.maximum(us + vt, 0.0)
                plsc.addupdate_scatter(num_v, [t16 + (c * N)], y * w)
            return carry

        lax.fori_loop(0, E // 16, edge_body, 0)

        pltpu.sync_copy(num_v, num_out.at[pl.ds(r0, CPW * N)])

        @pl.when(jnp.logical_or(wid == 0, wid == NW // 2))
        def _():
            pltpu.sync_copy(
                den_v, den_out.at[pl.ds(pl.multiple_of(head * N, N), N)])

    return sc_edge


def _layer_weights(layer, d_in):
    wu = jnp.concatenate([hp["f"]["W"][:, :d_in] for hp in layer], axis=0)
    bu = jnp.concatenate([hp["f"]["b"] for hp in layer], axis=0)
    wv = jnp.concatenate([hp["f"]["W"][:, d_in:] for hp in layer], axis=0)
    ws = jnp.concatenate(
        [hp["w"]["W"][:, :d_in] for hp in layer]
        + [hp["w"]["W"][:, d_in:] for hp in layer], axis=0)
    return wu, bu, wv, ws


def _run_sc(c, u, v, s, src, tgt):
    num, den = _make_sc_edge(c, c // 32)(
        u.reshape(c * N), v.reshape(c * N), s[:4].reshape(4 * N), src, tgt)
    return num.reshape(c, N), den.reshape(2, N)


def kernel(x, adj, src, tgt, Msrc, Mtgt, Mgraph, params):
    del adj, Msrc, Mtgt
    gat = params["gat"]
    dims = [(128, 32), (64, 64), (128, 64)]

    wu, bu, wv, ws = _layer_weights(gat[0], dims[0][0])
    c1 = 2 * dims[0][1]
    u, v, s = _tc_call(_proj0_body, [(c1, N), (c1, N), (8, N)],
                       (x, wu, bu, wv, ws))
    num, den = _run_sc(c1, u, v, s, src, tgt)

    for li in (1, 2):
        wu, bu, wv, ws = _layer_weights(gat[li], dims[li][0])
        cl = 2 * dims[li][1]
        u, v, s = _tc_call(_proj_mid_body, [(cl, N), (cl, N), (8, N)],
                           (num, den, wu, bu, wv, ws))
        num, den = _run_sc(cl, u, v, s, src, tgt)

    (out,) = _tc_call(
        _final_body, [(G, 10)],
        (num, den, Mgraph,
         params["mlp"][0]["W"], params["mlp"][0]["b"],
         params["mlp"][1]["W"], params["mlp"][1]["b"]))
    return out


# speedup vs baseline: 6.9475x; 1.8864x over previous
import functools

import jax
import jax.numpy as jnp
from jax import lax
from jax.experimental import pallas as pl
from jax.experimental.pallas import tpu as pltpu
from jax.experimental.pallas import tpu_sc as plsc

N = 2048
E = 16384
G = 16
EPS = 1e-6
F32 = jnp.float32


def _dot(a, b, dims):
    return lax.dot_general(a, b, (dims, ((), ())), preferred_element_type=F32)


def _fold_s(S):
    m1 = jnp.max(S[0:1, :]) + jnp.max(S[2:3, :])
    m2 = jnp.max(S[1:2, :]) + jnp.max(S[3:4, :])
    return jnp.concatenate(
        [S[0:1] - m1, S[1:2] - m2, S[2:4], jnp.zeros((4, N), F32)], axis=0)


def _proj0_body(x_ref, wu_ref, bu_ref, wv_ref, ws_ref, u_out, v_out, s_out):
    x = x_ref[...]
    u_out[...] = _dot(wu_ref[...], x, ((1,), (1,))) + bu_ref[...][:, None]
    v_out[...] = _dot(wv_ref[...], x, ((1,), (1,)))
    s_out[...] = _fold_s(_dot(ws_ref[...], x, ((1,), (1,))))


def _proj_mid_body(num_ref, den_ref, wu_ref, bu_ref, wv_ref, ws_ref,
                   u_out, v_out, s_out):
    num = num_ref[...]
    den = den_ref[...]
    half = num.shape[0] // 2
    hc = jnp.concatenate([num[:half] / (den[0:1] + EPS),
                          num[half:] / (den[1:2] + EPS)], axis=0)
    u_out[...] = _dot(wu_ref[...], hc, ((1,), (0,))) + bu_ref[...][:, None]
    v_out[...] = _dot(wv_ref[...], hc, ((1,), (0,)))
    s_out[...] = _fold_s(_dot(ws_ref[...], hc, ((1,), (0,))))


def _final_body(num_ref, den_ref, mg_ref, w1_ref, b1_ref, w2_ref, b2_ref,
                out_ref):
    num = num_ref[...]
    den = den_ref[...]
    half = num.shape[0] // 2
    hc = jnp.concatenate([num[:half] / (den[0:1] + EPS),
                          num[half:] / (den[1:2] + EPS)], axis=0)
    pooled = _dot(mg_ref[...], hc, ((0,), (1,)))
    z1 = jax.nn.relu(_dot(pooled, w1_ref[...], ((1,), (1,)))
                     + b1_ref[...][None, :])
    out_ref[...] = _dot(z1, w2_ref[...], ((1,), (1,))) + b2_ref[...][None, :]


def _tc_call(body, out_shapes, args):
    return pl.pallas_call(
        body,
        out_shape=[jax.ShapeDtypeStruct(s, F32) for s in out_shapes],
    )(*args)


@functools.cache
def _make_sc_edge(C, CPW):
    info = plsc.get_sparse_core_info()
    NC, NS = info.num_cores, info.num_subcores
    NW = NC * NS
    assert C == CPW * NW
    mesh = plsc.VectorSubcoreMesh(core_axis_name="c", subcore_axis_name="s")

    @functools.partial(
        pl.kernel, mesh=mesh,
        compiler_params=pltpu.CompilerParams(needs_layout_passes=False),
        out_type=[jax.ShapeDtypeStruct((C * N,), F32),
                  jax.ShapeDtypeStruct((2 * N,), F32)],
        scratch_types=[
            pltpu.VMEM((CPW * N,), F32),
            pltpu.VMEM((CPW * N,), F32),
            pltpu.VMEM((N,), F32),
            pltpu.VMEM((N,), F32),
            pltpu.VMEM((CPW * N,), F32),
            pltpu.VMEM((N,), F32),
            pltpu.VMEM((E,), jnp.int32),
            pltpu.VMEM((E,), jnp.int32),
        ],
    )
    def sc_edge(u_hbm, v_hbm, s_hbm, src_hbm, tgt_hbm, num_out, den_out,
                u_v, v_v, p_v, q_v, num_v, den_v, src_v, tgt_v):
        wid = lax.axis_index("s") * NC + lax.axis_index("c")
        head = wid // (NW // 2)
        r0 = pl.multiple_of(wid * (CPW * N), CPW * N)

        pltpu.sync_copy(u_hbm.at[pl.ds(r0, CPW * N)], u_v)
        pltpu.sync_copy(v_hbm.at[pl.ds(r0, CPW * N)], v_v)
        pltpu.sync_copy(s_hbm.at[pl.ds(pl.multiple_of(head * N, N), N)], p_v)
        pltpu.sync_copy(
            s_hbm.at[pl.ds(pl.multiple_of((2 + head) * N, N), N)], q_v)
        pltpu.sync_copy(src_hbm, src_v)
        pltpu.sync_copy(tgt_hbm, tgt_v)

        zf = jnp.zeros((16,), F32)

        @plsc.parallel_loop(0, CPW * N // 16, 1, unroll=8)
        def zero_num(j):
            num_v[pl.ds(pl.multiple_of(j * 16, 16), 16)] = zf

        @plsc.parallel_loop(0, N // 16, 1, unroll=8)
        def zero_den(j):
            den_v[pl.ds(pl.multiple_of(j * 16, 16), 16)] = zf

        @plsc.parallel_loop(0, E // 16, 1, unroll=4)
        def edge_body(g):
            base = pl.multiple_of(g * 16, 16)
            s16 = src_v[pl.ds(base, 16)]
            t16 = tgt_v[pl.ds(base, 16)]
            ps = plsc.load_gather(p_v, [s16])
            qt = plsc.load_gather(q_v, [t16])
            w = jnp.exp(ps + qt)
            plsc.addupdate_scatter(den_v, [t16], w)
            for c in range(CPW):
                us = plsc.load_gather(u_v, [s16 + (c * N)])
                vt = plsc.load_gather(v_v, [t16 + (c * N)])
                y = jnp.maximum(us + vt, 0.0)
                plsc.addupdate_scatter(num_v, [t16 + (c * N)], y * w)

        pltpu.sync_copy(num_v, num_out.at[pl.ds(r0, CPW * N)])

        @pl.when(jnp.logical_or(wid == 0, wid == NW // 2))
        def _():
            pltpu.sync_copy(
                den_v, den_out.at[pl.ds(pl.multiple_of(head * N, N), N)])

    return sc_edge


def _layer_weights(layer, d_in):
    wu = jnp.concatenate([hp["f"]["W"][:, :d_in] for hp in layer], axis=0)
    bu = jnp.concatenate([hp["f"]["b"] for hp in layer], axis=0)
    wv = jnp.concatenate([hp["f"]["W"][:, d_in:] for hp in layer], axis=0)
    ws = jnp.concatenate(
        [hp["w"]["W"][:, :d_in] for hp in layer]
        + [hp["w"]["W"][:, d_in:] for hp in layer], axis=0)
    return wu, bu, wv, ws


def _run_sc(c, u, v, s, src, tgt):
    num, den = _make_sc_edge(c, c // 32)(
        u.reshape(c * N), v.reshape(c * N), s[:4].reshape(4 * N), src, tgt)
    return num.reshape(c, N), den.reshape(2, N)


def kernel(x, adj, src, tgt, Msrc, Mtgt, Mgraph, params):
    del adj, Msrc, Mtgt
    gat = params["gat"]
    dims = [(128, 32), (64, 64), (128, 64)]

    wu, bu, wv, ws = _layer_weights(gat[0], dims[0][0])
    c1 = 2 * dims[0][1]
    u, v, s = _tc_call(_proj0_body, [(c1, N), (c1, N), (8, N)],
                       (x, wu, bu, wv, ws))
    num, den = _run_sc(c1, u, v, s, src, tgt)

    for li in (1, 2):
        wu, bu, wv, ws = _layer_weights(gat[li], dims[li][0])
        cl = 2 * dims[li][1]
        u, v, s = _tc_call(_proj_mid_body, [(cl, N), (cl, N), (8, N)],
                           (num, den, wu, bu, wv, ws))
        num, den = _run_sc(cl, u, v, s, src, tgt)

    (out,) = _tc_call(
        _final_body, [(G, 10)],
        (num, den, Mgraph,
         params["mlp"][0]["W"], params["mlp"][0]["b"],
         params["mlp"][1]["W"], params["mlp"][1]["b"]))
    return out
